# Initial kernel scaffold; baseline (speedup 1.0000x reference)
import jax
import jax.numpy as jnp
from jax.experimental import pallas as pl


def kernel(x_cat, edge_index, edge_feats, edge_vec, emb_tables, eW1, eb1, eW2, eb2, fc0W1, fc0b1, fc0W2, fc0b2, fc1W1, fc1b1, fc1W2, fc1b2):
    raise NotImplementedError("write your pallas kernel here")


# speedup vs baseline: 49.4064x; 49.4064x over previous
import functools
import math

import jax
import jax.numpy as jnp
from jax import lax
from jax.experimental import pallas as pl
from jax.experimental.pallas import tpu as pltpu
from jax.experimental.pallas import tpu_sc as plsc

F32 = jnp.float32
_NS = 16
_NV = 4
_NCAT = 10
_DEDIM = 32

_NC = 2
_NSUB = 16
_NW = _NC * _NSUB
_CH = 80
_G = 5


def _node_body(xcat_ref, emb_ref, out_ref):
    x = xcat_ref[...]
    rows = x.shape[0]
    iota = lax.broadcasted_iota(jnp.int32, (rows, _NS), 1)
    acc = jnp.zeros((rows, _NS), F32)
    for i in range(_NCAT):
        oh = jnp.where(x[:, i:i + 1] == iota, 1.0, 0.0).astype(F32)
        acc = acc + jnp.dot(oh, emb_ref[i], preferred_element_type=F32)
    out_ref[...] = acc


def _edge_common(feats_ref, vec_ref, eW1_ref, eb1_ref, eW2_ref, eb2_ref):
    vec = vec_ref[...]
    feats = feats_ref[...]
    d2 = jnp.sum(vec * vec, axis=1, keepdims=True)
    r = jnp.sqrt(d2 + 1e-12)
    step = 5.0 / (_DEDIM - 1)
    off = lax.broadcasted_iota(jnp.int32, (1, _DEDIM), 1).astype(F32) * step
    coeff = -0.5 / step ** 2
    demb = jnp.exp(coeff * (r - off) ** 2)
    ea_in = jnp.concatenate([feats, demb], axis=1)
    h = jnp.maximum(
        jnp.dot(ea_in, eW1_ref[...], preferred_element_type=F32) + eb1_ref[...], 0.0)
    ea = jnp.dot(h, eW2_ref[...], preferred_element_type=F32) + eb2_ref[...]
    sh1 = jnp.sqrt(3.0) * (vec / r)
    return ea, sh1


def _edge0_body(feats_ref, vec_ref, gs_ref, gd_ref,
                eW1_ref, eb1_ref, eW2_ref, eb2_ref,
                W1_ref, b1_ref, W2_ref, b2_ref, out_ref):
    ea, sh1 = _edge_common(feats_ref, vec_ref, eW1_ref, eb1_ref, eW2_ref, eb2_ref)
    gd = gd_ref[...]
    ef = jnp.concatenate([ea, gs_ref[...], gd], axis=1)
    h0 = jnp.maximum(
        jnp.dot(ef, W1_ref[...], preferred_element_type=F32) + b1_ref[...], 0.0)
    w = jnp.dot(h0, W2_ref[...], preferred_element_type=F32) + b2_ref[...]
    rows = gd.shape[0]
    acc16 = jnp.zeros((rows, _NS), F32)
    acc4 = jnp.zeros((rows, _NV), F32)
    for i in range(_NS):
        xi = gd[:, i:i + 1]
        acc16 = acc16 + w[:, 16 * i:16 * i + 16] * xi
        acc4 = acc4 + w[:, 256 + 4 * i:256 + 4 * i + 4] * xi
    inv = 1.0 / math.sqrt(float(_NS))
    out_ref[:, 0:16] = acc16 * inv
    for k in range(_NV):
        out_ref[:, 16 + 3 * k:19 + 3 * k] = (acc4[:, k:k + 1] * sh1) * inv
    out_ref[:, 28:29] = jnp.ones((rows, 1), F32)
    out_ref[:, 29:32] = jnp.zeros((rows, 3), F32)


def _edge1_body(feats_ref, vec_ref, gs_ref, gd_ref,
                eW1_ref, eb1_ref, eW2_ref, eb2_ref,
                W1_ref, b1_ref, W2_ref, b2_ref, out_ref):
    ea, sh1 = _edge_common(feats_ref, vec_ref, eW1_ref, eb1_ref, eW2_ref, eb2_ref)
    gd = gd_ref[...]
    xs = gd[:, 0:16]
    ef = jnp.concatenate([ea, gs_ref[...], xs], axis=1)
    h1 = jnp.maximum(
        jnp.dot(ef, W1_ref[...], preferred_element_type=F32) + b1_ref[...], 0.0)
    w = jnp.dot(h1, W2_ref[...], preferred_element_type=F32) + b2_ref[...]
    rows = gd.shape[0]
    acc_a = jnp.zeros((rows, _NS), F32)
    for i in range(_NS):
        acc_a = acc_a + w[:, 16 * i:16 * i + 16] * xs[:, i:i + 1]
    acc_b = jnp.zeros((rows, _NS), F32)
    for i in range(_NV):
        si = jnp.sum(gd[:, 16 + 3 * i:19 + 3 * i] * sh1, axis=1, keepdims=True)
        acc_b = acc_b + w[:, 256 + 16 * i:256 + 16 * i + 16] * si
    out_ref[...] = (acc_a + acc_b * (1.0 / math.sqrt(3.0))) * (
        1.0 / math.sqrt(float(_NS + _NV)))


def _div0_body(part_ref, node16_ref, node32_ref, cnt_ref):
    p = part_ref[0] + part_ref[1]
    cnt = jnp.maximum(p[:, 28:29], 1.0)
    n32 = p / cnt
    node32_ref[...] = n32
    node16_ref[...] = n32[:, 0:16]
    cnt_ref[...] = cnt


def _div1_body(part_ref, cnt_ref, out_ref):
    p = part_ref[0] + part_ref[1]
    out_ref[...] = p / cnt_ref[...]


def _make_gather(N, E, D1, D2):
    PW = E // _NW
    NCH = PW // _CH
    NG = NCH // _G
    EG = _G * _CH
    assert PW * _NW == E and NCH * _CH == PW and NG * _G == NCH
    mesh = plsc.VectorSubcoreMesh(core_axis_name="c", subcore_axis_name="s")

    @functools.partial(
        pl.kernel,
        out_type=(jax.ShapeDtypeStruct((E, D1), F32),
                  jax.ShapeDtypeStruct((E, D2), F32)),
        mesh=mesh,
        scratch_types=[
            pltpu.VMEM((NCH, _CH), jnp.int32),
            pltpu.VMEM((NCH, _CH), jnp.int32),
            pltpu.VMEM((EG, D1), F32),
            pltpu.VMEM((EG, D2), F32),
            pltpu.SemaphoreType.DMA,
        ],
        compiler_params=pltpu.CompilerParams(use_tc_tiling_on_sc=False),
    )
    def gath(tabs_hbm, tabd_hbm, src3_hbm, dst3_hbm, outs_hbm, outd_hbm,
             idxs, idxd, rows_s, rows_d, sem):
        wid = lax.axis_index("s") * _NC + lax.axis_index("c")
        pltpu.sync_copy(src3_hbm.at[wid], idxs)
        pltpu.sync_copy(dst3_hbm.at[wid], idxd)
        ebase = wid * PW

        def grp(g, carry):
            cps = []
            for k in range(_G):
                j = g * _G + k
                cps.append(pltpu.async_copy(
                    tabs_hbm.at[idxs.at[j]], rows_s.at[pl.ds(k * _CH, _CH)], sem))
                cps.append(pltpu.async_copy(
                    tabd_hbm.at[idxd.at[j]], rows_d.at[pl.ds(k * _CH, _CH)], sem))
            for cp in cps:
                cp.wait()
            pltpu.sync_copy(rows_s, outs_hbm.at[pl.ds(ebase + g * EG, EG)])
            pltpu.sync_copy(rows_d, outd_hbm.at[pl.ds(ebase + g * EG, EG)])
            return carry

        lax.fori_loop(0, NG, grp, 0)

    return gath


def _make_scatter(N, E, D):
    PW = E // _NW
    NCH = PW // _CH
    NG = NCH // _G
    EG = _G * _CH
    mesh = plsc.VectorSubcoreMesh(core_axis_name="c", subcore_axis_name="s")

    @functools.partial(
        pl.kernel,
        out_type=jax.ShapeDtypeStruct((_NC, N, D), F32),
        mesh=mesh,
        scratch_types=[
            pltpu.VMEM((NCH, _CH), jnp.int32),
            pltpu.VMEM((EG, D), F32),
            pltpu.VMEM_SHARED((N, D), F32),
            pltpu.SemaphoreType.DMA,
        ],
        compiler_params=pltpu.CompilerParams(use_tc_tiling_on_sc=False),
    )
    def scat(tp_hbm, src3_hbm, zero_hbm, out_hbm, idxs, rows, acc, sem):
        c = lax.axis_index("c")
        s = lax.axis_index("s")

        @pl.when(s == 0)
        def _init():
            pltpu.sync_copy(zero_hbm, acc)

        plsc.subcore_barrier()
        wid = s * _NC + c
        pltpu.sync_copy(src3_hbm.at[wid], idxs)
        ebase = wid * PW

        def grp(g, carry):
            pltpu.sync_copy(tp_hbm.at[pl.ds(ebase + g * EG, EG)], rows)
            cps = []
            for k in range(_G):
                j = g * _G + k
                cps.append(pltpu.async_copy(
                    rows.at[pl.ds(k * _CH, _CH)], acc.at[idxs.at[j]], sem,
                    add=True))
            for cp in cps:
                cp.wait()
            return carry

        lax.fori_loop(0, NG, grp, 0)
        plsc.subcore_barrier()

        @pl.when(s == 0)
        def _out():
            pltpu.sync_copy(acc, out_hbm.at[c])

    return scat


def _full(shape):
    return pl.BlockSpec(shape, lambda i: tuple(0 for _ in shape))


def kernel(x_cat, edge_index, edge_feats, edge_vec, emb_tables,
           eW1, eb1, eW2, eb2,
           fc0W1, fc0b1, fc0W2, fc0b2,
           fc1W1, fc1b1, fc1W2, fc1b2):
    N = x_cat.shape[0]
    E = edge_index.shape[1]
    src = edge_index[0].astype(jnp.int32)
    dst = edge_index[1].astype(jnp.int32)
    PW = E // _NW
    src2 = src.reshape(_NW, PW // _CH, _CH)
    dst2 = dst.reshape(_NW, PW // _CH, _CH)
    x_cat = x_cat.astype(jnp.int32)
    eb1r = eb1.reshape(1, -1)
    eb2r = eb2.reshape(1, -1)
    fc0b1r = fc0b1.reshape(1, -1)
    fc0b2r = fc0b2.reshape(1, -1)
    fc1b1r = fc1b1.reshape(1, -1)
    fc1b2r = fc1b2.reshape(1, -1)

    RN = 2000
    node = pl.pallas_call(
        _node_body,
        grid=(N // RN,),
        in_specs=[pl.BlockSpec((RN, _NCAT), lambda i: (i, 0)),
                  _full((_NCAT, 16, _NS))],
        out_specs=pl.BlockSpec((RN, _NS), lambda i: (i, 0)),
        out_shape=jax.ShapeDtypeStruct((N, _NS), F32),
    )(x_cat, emb_tables)

    g0s, g0d = _make_gather(N, E, _NS, _NS)(node, node, src2, dst2)

    B = 2000
    edge_specs = [pl.BlockSpec((B, 4), lambda i: (i, 0)),
                  pl.BlockSpec((B, 3), lambda i: (i, 0))]
    w_specs0 = [_full((36, _NS)), _full((1, _NS)), _full((_NS, _NS)), _full((1, _NS)),
                _full((48, 48)), _full((1, 48)), _full((48, 320)), _full((1, 320))]
    tp0 = pl.pallas_call(
        _edge0_body,
        grid=(E // B,),
        in_specs=edge_specs + [pl.BlockSpec((B, _NS), lambda i: (i, 0)),
                               pl.BlockSpec((B, _NS), lambda i: (i, 0))] + w_specs0,
        out_specs=pl.BlockSpec((B, 32), lambda i: (i, 0)),
        out_shape=jax.ShapeDtypeStruct((E, 32), F32),
    )(edge_feats, edge_vec, g0s, g0d,
      eW1, eb1r, eW2, eb2r, fc0W1, fc0b1r, fc0W2, fc0b2r)

    part0 = _make_scatter(N, E, 32)(tp0, src2, jnp.zeros((N, 32), F32))

    RD = 2000
    node16, node32, cnt = pl.pallas_call(
        _div0_body,
        grid=(N // RD,),
        in_specs=[pl.BlockSpec((_NC, RD, 32), lambda i: (0, i, 0))],
        out_specs=[pl.BlockSpec((RD, _NS), lambda i: (i, 0)),
                   pl.BlockSpec((RD, 32), lambda i: (i, 0)),
                   pl.BlockSpec((RD, 1), lambda i: (i, 0))],
        out_shape=[jax.ShapeDtypeStruct((N, _NS), F32),
                   jax.ShapeDtypeStruct((N, 32), F32),
                   jax.ShapeDtypeStruct((N, 1), F32)],
    )(part0)

    g1s, g1d = _make_gather(N, E, _NS, 32)(node16, node32, src2, dst2)

    tp1 = pl.pallas_call(
        _edge1_body,
        grid=(E // B,),
        in_specs=edge_specs + [pl.BlockSpec((B, _NS), lambda i: (i, 0)),
                               pl.BlockSpec((B, 32), lambda i: (i, 0))] + w_specs0,
        out_specs=pl.BlockSpec((B, _NS), lambda i: (i, 0)),
        out_shape=jax.ShapeDtypeStruct((E, _NS), F32),
    )(edge_feats, edge_vec, g1s, g1d,
      eW1, eb1r, eW2, eb2r, fc1W1, fc1b1r, fc1W2, fc1b2r)

    part1 = _make_scatter(N, E, _NS)(tp1, src2, jnp.zeros((N, _NS), F32))

    node2 = pl.pallas_call(
        _div1_body,
        grid=(N // RD,),
        in_specs=[pl.BlockSpec((_NC, RD, _NS), lambda i: (0, i, 0)),
                  pl.BlockSpec((RD, 1), lambda i: (i, 0))],
        out_specs=pl.BlockSpec((RD, _NS), lambda i: (i, 0)),
        out_shape=jax.ShapeDtypeStruct((N, _NS), F32),
    )(part1, cnt)
    return node2
